# baseline (speedup 1.0000x reference)
import functools

import jax
import jax.numpy as jnp
from jax import lax
from jax.experimental import pallas as pl
from jax.experimental.pallas import tpu as pltpu
from jax.experimental.pallas import tpu_sc as plsc

_B = 128
_NU = 10000
_NI = 2000
_D = 64
_P = _NI + _D
_W = _D + _NU + _D + _NI
_ITEM_BLK0 = 9984
_PPAD = 2176
_NW = 32
_BPW = _B // _NW


def _compute(uid_ref, iid_ref, uf_ref, if_ref, umem_ref, imem_ref,
             uw_ref, uwl_ref, ub_ref, iw_ref, iwl_ref, ib_ref,
             twt_ref, tb_ref, pw_ref, pb_ref,
             new_u_ref, prev_u_ref, new_i_ref, prev_i_ref, pd_ref,
             pu_s, pi_s, wu_s, wi_s, sem_gu, sem_gi, sem_w):
    gu = []
    gi = []
    for b in range(_B):
        u = uid_ref[b]
        i = iid_ref[b]
        cu = pltpu.make_async_copy(
            umem_ref.at[b].at[pl.ds(u, 1), :], pu_s.at[pl.ds(b, 1), :], sem_gu)
        ci = pltpu.make_async_copy(
            imem_ref.at[b].at[pl.ds(i, 1), :], pi_s.at[pl.ds(b, 1), :], sem_gi)
        cu.start()
        ci.start()
        gu.append(cu)
        gi.append(ci)

    wcu = pltpu.make_async_copy(pw_ref.at[:, pl.ds(0, 128)], wu_s, sem_w)
    wci = pltpu.make_async_copy(pw_ref.at[:, pl.ds(_ITEM_BLK0, 256)], wi_s,
                                sem_w)
    wcu.start()
    wci.start()

    for c in gu:
        c.wait()
    for c in gi:
        c.wait()

    prev_u = pu_s[...]
    prev_i = pi_s[...]
    prev_u_ref[...] = prev_u
    prev_i_ref[...] = prev_i

    uf = uf_ref[...]
    itf = if_ref[...]
    time_context = uf * twt_ref[...] + tb_ref[...]
    user_proj = (1.0 + time_context) * prev_u

    f32 = jnp.float32
    dn = (((1,), (1,)), ((), ()))
    uw = uw_ref[...]
    iw = iw_ref[...]
    u_pre = (lax.dot_general(prev_u, uw[:, 0:_D], dn, preferred_element_type=f32)
             + lax.dot_general(prev_i, uw[:, _D:2 * _D], dn,
                               preferred_element_type=f32)
             + uf * uwl_ref[...] + ub_ref[...])
    i_pre = (lax.dot_general(prev_i, iw[:, 0:_D], dn, preferred_element_type=f32)
             + lax.dot_general(prev_u, iw[:, _D:2 * _D], dn,
                               preferred_element_type=f32)
             + itf * iwl_ref[...] + ib_ref[...])
    new_u_ref[...] = jax.nn.sigmoid(u_pre)
    new_i_ref[...] = jax.nn.sigmoid(i_pre)

    wcu.wait()
    wci.wait()
    pd_ref[...] = (
        lax.dot_general(user_proj, wu_s[...][:, 0:_D], dn,
                        preferred_element_type=f32)
        + lax.dot_general(prev_i, wi_s[...][:, 80:144], dn,
                          preferred_element_type=f32)
        + pb_ref[...])


def _copy_scatter(ids_ref, mem_ref, new_ref, out_ref):
    b = pl.program_id(0)
    out_ref[...] = mem_ref[...]
    off = ids_ref[b] * _D
    s = off % 128
    aoff = pl.multiple_of(off - s, 128)
    row = new_ref[b, :]
    doubled = jnp.concatenate([row, row])
    lane = lax.iota(jnp.int32, 128)
    mask = (lane >= s) & (lane < s + _D)
    seg = out_ref[0, 0, pl.ds(aoff, 128)]
    out_ref[0, 0, pl.ds(aoff, 128)] = jnp.where(mask, doubled, seg)


def _scatter_copy_call(mem, ids, new_rows, n):
    f32 = jnp.float32
    flat = pl.pallas_call(
        _copy_scatter,
        grid=(_B,),
        in_specs=[
            pl.BlockSpec(memory_space=pltpu.MemorySpace.SMEM),
            pl.BlockSpec((1, 1, n * _D), lambda b: (b, 0, 0)),
            pl.BlockSpec((_B, _D), lambda b: (0, 0)),
        ],
        out_specs=pl.BlockSpec((1, 1, n * _D), lambda b: (b, 0, 0)),
        out_shape=jax.ShapeDtypeStruct((_B, 1, n * _D), f32),
    )(ids, mem.reshape(_B, 1, n * _D), new_rows)
    return flat.reshape(_B, n, _D)


def _sc_cols(pwflat_ref, cols_ref, pd_ref, out_ref,
             cols_v, idx_v, vals_v, pd_v, out_v, sem):
    i32 = jnp.int32
    wid = lax.axis_index("c") * 16 + lax.axis_index("s")
    b0 = wid * _BPW
    pltpu.sync_copy(cols_ref.at[pl.ds(2 * b0, 2 * _BPW)], cols_v)
    pltpu.sync_copy(pd_ref.at[pl.ds(b0, _BPW)], pd_v)

    for k in range(2 * _BPW):
        col = cols_v[k]

        def fill(t, _, k=k, col=col):
            jv = lax.iota(i32, 16) + 16 * t
            idx = jnp.where(jv < _P, jv * _W + col, 0)
            idx_v[k, pl.ds(16 * t, 16)] = idx
            return 0

        lax.fori_loop(0, _PPAD // 16, fill, 0)

    copies = []
    for k in range(2 * _BPW):
        for j in range(_PPAD // 128):
            c = pltpu.make_async_copy(
                pwflat_ref.at[idx_v.at[k, pl.ds(128 * j, 128)]],
                vals_v.at[k, pl.ds(128 * j, 128)], sem)
            c.start()
            copies.append(c)
    for c in copies:
        c.wait()

    for bl in range(_BPW):
        def acc(t, _, bl=bl):
            s = pl.ds(16 * t, 16)
            out_v[bl, s] = pd_v[bl, s] + vals_v[2 * bl, s] + vals_v[2 * bl + 1, s]
            return 0

        lax.fori_loop(0, _P // 16, acc, 0)
    pltpu.sync_copy(out_v, out_ref.at[pl.ds(b0, _BPW)])


def kernel(user_ids, item_ids, user_features, item_features, user_memory,
           item_memory, user_rnn_w, user_rnn_b, item_rnn_w, item_rnn_b,
           time_w, time_b, pred_w, pred_b):
    f32 = jnp.float32
    smem = pl.BlockSpec(memory_space=pltpu.MemorySpace.SMEM)
    vmem = pl.BlockSpec(memory_space=pltpu.MemorySpace.VMEM)
    hbm = pl.BlockSpec(memory_space=pltpu.MemorySpace.HBM)

    uwl = user_rnn_w[:, 2 * _D].reshape(1, _D)
    iwl = item_rnn_w[:, 2 * _D].reshape(1, _D)
    twt = time_w.reshape(1, _D)
    tb2 = time_b.reshape(1, _D)
    ub2 = user_rnn_b.reshape(1, _D)
    ib2 = item_rnn_b.reshape(1, _D)
    pb2 = pred_b.reshape(1, _P)

    new_u, prev_u, new_i, prev_i, pred_dense = pl.pallas_call(
        _compute,
        grid_spec=pltpu.PrefetchScalarGridSpec(
            num_scalar_prefetch=0,
            in_specs=[smem, smem, vmem, vmem, hbm, hbm,
                      vmem, vmem, vmem, vmem, vmem, vmem, vmem, vmem,
                      hbm, vmem],
            out_specs=[vmem, vmem, vmem, vmem, vmem],
            scratch_shapes=[
                pltpu.VMEM((_B, _D), f32),
                pltpu.VMEM((_B, _D), f32),
                pltpu.VMEM((_P, 128), f32),
                pltpu.VMEM((_P, 256), f32),
                pltpu.SemaphoreType.DMA,
                pltpu.SemaphoreType.DMA,
                pltpu.SemaphoreType.DMA,
            ],
        ),
        out_shape=(
            jax.ShapeDtypeStruct((_B, _D), f32),
            jax.ShapeDtypeStruct((_B, _D), f32),
            jax.ShapeDtypeStruct((_B, _D), f32),
            jax.ShapeDtypeStruct((_B, _D), f32),
            jax.ShapeDtypeStruct((_B, _P), f32),
        ),
    )(user_ids, item_ids, user_features, item_features, user_memory,
      item_memory, user_rnn_w, uwl, ub2, item_rnn_w, iwl, ib2, twt, tb2,
      pred_w, pb2)

    new_umem = _scatter_copy_call(user_memory, user_ids, new_u, _NU)
    new_imem = _scatter_copy_call(item_memory, item_ids, new_i, _NI)

    colvals = jnp.stack([user_ids + _D, item_ids + (2 * _D + _NU)],
                        axis=1).reshape(2 * _B)
    cols_pre = jnp.broadcast_to(colvals[:, None], (2 * _B, 16))

    predicted = pl.kernel(
        _sc_cols,
        out_type=jax.ShapeDtypeStruct((_B, _P), f32),
        mesh=plsc.VectorSubcoreMesh(core_axis_name="c", subcore_axis_name="s",
                                    num_cores=2, num_subcores=16),
        scratch_types=[
            pltpu.VMEM((2 * _BPW, 16), jnp.int32),
            pltpu.VMEM((2 * _BPW, _PPAD), jnp.int32),
            pltpu.VMEM((2 * _BPW, _PPAD), f32),
            pltpu.VMEM((_BPW, _P), f32),
            pltpu.VMEM((_BPW, _P), f32),
            pltpu.SemaphoreType.DMA,
        ],
    )(pred_w.reshape(-1), cols_pre, pred_dense)

    return (new_u, prev_u, new_i, predicted, prev_i, new_umem, new_imem)


# speedup vs baseline: 1.4971x; 1.4971x over previous
import functools

import jax
import jax.numpy as jnp
from jax import lax
from jax.experimental import pallas as pl
from jax.experimental.pallas import tpu as pltpu
from jax.experimental.pallas import tpu_sc as plsc

_B = 128
_NU = 10000
_NI = 2000
_D = 64
_P = _NI + _D
_W = _D + _NU + _D + _NI
_ITEM_BLK0 = 9984
_PPAD = 2176
_NW = 32
_BPW = _B // _NW


def _compute(uid_ref, iid_ref, uf_ref, if_ref, umem_ref, imem_ref,
             uw_ref, uwl_ref, ub_ref, iw_ref, iwl_ref, ib_ref,
             twt_ref, tb_ref, pw_ref, pb_ref,
             new_u_ref, prev_u_ref, new_i_ref, prev_i_ref, pd_ref,
             pu_s, pi_s, wu_s, wi_s, sem_gu, sem_gi, sem_w):
    gu = []
    gi = []
    for b in range(_B):
        u = uid_ref[b]
        i = iid_ref[b]
        cu = pltpu.make_async_copy(
            umem_ref.at[b].at[pl.ds(u, 1), :], pu_s.at[pl.ds(b, 1), :], sem_gu)
        ci = pltpu.make_async_copy(
            imem_ref.at[b].at[pl.ds(i, 1), :], pi_s.at[pl.ds(b, 1), :], sem_gi)
        cu.start()
        ci.start()
        gu.append(cu)
        gi.append(ci)

    wcu = pltpu.make_async_copy(pw_ref.at[:, pl.ds(0, 128)], wu_s, sem_w)
    wci = pltpu.make_async_copy(pw_ref.at[:, pl.ds(_ITEM_BLK0, 256)], wi_s,
                                sem_w)
    wcu.start()
    wci.start()

    for c in gu:
        c.wait()
    for c in gi:
        c.wait()

    prev_u = pu_s[...]
    prev_i = pi_s[...]
    prev_u_ref[...] = prev_u
    prev_i_ref[...] = prev_i

    uf = uf_ref[...]
    itf = if_ref[...]
    time_context = uf * twt_ref[...] + tb_ref[...]
    user_proj = (1.0 + time_context) * prev_u

    f32 = jnp.float32
    dn = (((1,), (1,)), ((), ()))
    uw = uw_ref[...]
    iw = iw_ref[...]
    u_pre = (lax.dot_general(prev_u, uw[:, 0:_D], dn, preferred_element_type=f32)
             + lax.dot_general(prev_i, uw[:, _D:2 * _D], dn,
                               preferred_element_type=f32)
             + uf * uwl_ref[...] + ub_ref[...])
    i_pre = (lax.dot_general(prev_i, iw[:, 0:_D], dn, preferred_element_type=f32)
             + lax.dot_general(prev_u, iw[:, _D:2 * _D], dn,
                               preferred_element_type=f32)
             + itf * iwl_ref[...] + ib_ref[...])
    new_u_ref[...] = jax.nn.sigmoid(u_pre)
    new_i_ref[...] = jax.nn.sigmoid(i_pre)

    wcu.wait()
    wci.wait()
    pd_ref[...] = (
        lax.dot_general(user_proj, wu_s[...][:, 0:_D], dn,
                        preferred_element_type=f32)
        + lax.dot_general(prev_i, wi_s[...][:, 80:144], dn,
                          preferred_element_type=f32)
        + pb_ref[...])


def _copy_scatter(ids_ref, mem_ref, new_ref, out_ref):
    b = pl.program_id(0)
    out_ref[...] = mem_ref[...]
    rid = ids_ref[b]
    r0 = rid // 2
    lane_s = (rid % 2) * _D
    a8 = pl.multiple_of(r0 - r0 % 8, 8)
    sub = r0 % 8
    row = new_ref[b, :]
    doubled = jnp.concatenate([row, row])
    i32 = jnp.int32
    sl = lax.broadcasted_iota(i32, (8, 128), 0)
    ln = lax.broadcasted_iota(i32, (8, 128), 1)
    mask = (sl == sub) & (ln >= lane_s) & (ln < lane_s + _D)
    seg = out_ref[0, pl.ds(a8, 8), :]
    val = jnp.broadcast_to(doubled[None, :], (8, 128))
    out_ref[0, pl.ds(a8, 8), :] = jnp.where(mask, val, seg)


def _scatter_copy_call(mem, ids, new_rows, n):
    f32 = jnp.float32
    nr = n * _D // 128
    flat = pl.pallas_call(
        _copy_scatter,
        grid=(_B,),
        in_specs=[
            pl.BlockSpec(memory_space=pltpu.MemorySpace.SMEM),
            pl.BlockSpec((1, nr, 128), lambda b: (b, 0, 0)),
            pl.BlockSpec((_B, _D), lambda b: (0, 0)),
        ],
        out_specs=pl.BlockSpec((1, nr, 128), lambda b: (b, 0, 0)),
        out_shape=jax.ShapeDtypeStruct((_B, nr, 128), f32),
    )(ids, mem.reshape(_B, nr, 128), new_rows)
    return flat.reshape(_B, n, _D)


def _sc_cols(pwflat_ref, cols_ref, pd_ref, out_ref,
             cols_v, idx_v, vals_v, pd_v, out_v, sem):
    i32 = jnp.int32
    wid = lax.axis_index("c") * 16 + lax.axis_index("s")
    b0 = wid * _BPW
    pltpu.sync_copy(cols_ref.at[pl.ds(2 * b0, 2 * _BPW)], cols_v)
    pltpu.sync_copy(pd_ref.at[pl.ds(b0, _BPW)], pd_v)

    for k in range(2 * _BPW):
        col = cols_v[k]

        def fill(t, _, k=k, col=col):
            jv = lax.iota(i32, 16) + 16 * t
            idx = jnp.where(jv < _P, jv * _W + col, 0)
            idx_v[k, pl.ds(16 * t, 16)] = idx
            return 0

        lax.fori_loop(0, _PPAD // 16, fill, 0)

    copies = []
    for k in range(2 * _BPW):
        for j in range(_PPAD // 128):
            c = pltpu.make_async_copy(
                pwflat_ref.at[idx_v.at[k, pl.ds(128 * j, 128)]],
                vals_v.at[k, pl.ds(128 * j, 128)], sem)
            c.start()
            copies.append(c)
    for c in copies:
        c.wait()

    for bl in range(_BPW):
        def acc(t, _, bl=bl):
            s = pl.ds(16 * t, 16)
            out_v[bl, s] = pd_v[bl, s] + vals_v[2 * bl, s] + vals_v[2 * bl + 1, s]
            return 0

        lax.fori_loop(0, _P // 16, acc, 0)
    pltpu.sync_copy(out_v, out_ref.at[pl.ds(b0, _BPW)])


def kernel(user_ids, item_ids, user_features, item_features, user_memory,
           item_memory, user_rnn_w, user_rnn_b, item_rnn_w, item_rnn_b,
           time_w, time_b, pred_w, pred_b):
    f32 = jnp.float32
    smem = pl.BlockSpec(memory_space=pltpu.MemorySpace.SMEM)
    vmem = pl.BlockSpec(memory_space=pltpu.MemorySpace.VMEM)
    hbm = pl.BlockSpec(memory_space=pltpu.MemorySpace.HBM)

    uwl = user_rnn_w[:, 2 * _D].reshape(1, _D)
    iwl = item_rnn_w[:, 2 * _D].reshape(1, _D)
    twt = time_w.reshape(1, _D)
    tb2 = time_b.reshape(1, _D)
    ub2 = user_rnn_b.reshape(1, _D)
    ib2 = item_rnn_b.reshape(1, _D)
    pb2 = pred_b.reshape(1, _P)

    new_u, prev_u, new_i, prev_i, pred_dense = pl.pallas_call(
        _compute,
        grid_spec=pltpu.PrefetchScalarGridSpec(
            num_scalar_prefetch=0,
            in_specs=[smem, smem, vmem, vmem, hbm, hbm,
                      vmem, vmem, vmem, vmem, vmem, vmem, vmem, vmem,
                      hbm, vmem],
            out_specs=[vmem, vmem, vmem, vmem, vmem],
            scratch_shapes=[
                pltpu.VMEM((_B, _D), f32),
                pltpu.VMEM((_B, _D), f32),
                pltpu.VMEM((_P, 128), f32),
                pltpu.VMEM((_P, 256), f32),
                pltpu.SemaphoreType.DMA,
                pltpu.SemaphoreType.DMA,
                pltpu.SemaphoreType.DMA,
            ],
        ),
        out_shape=(
            jax.ShapeDtypeStruct((_B, _D), f32),
            jax.ShapeDtypeStruct((_B, _D), f32),
            jax.ShapeDtypeStruct((_B, _D), f32),
            jax.ShapeDtypeStruct((_B, _D), f32),
            jax.ShapeDtypeStruct((_B, _P), f32),
        ),
    )(user_ids, item_ids, user_features, item_features, user_memory,
      item_memory, user_rnn_w, uwl, ub2, item_rnn_w, iwl, ib2, twt, tb2,
      pred_w, pb2)

    new_umem = _scatter_copy_call(user_memory, user_ids, new_u, _NU)
    new_imem = _scatter_copy_call(item_memory, item_ids, new_i, _NI)

    colvals = jnp.stack([user_ids + _D, item_ids + (2 * _D + _NU)],
                        axis=1).reshape(2 * _B)
    cols_pre = jnp.broadcast_to(colvals[:, None], (2 * _B, 16))

    predicted = pl.kernel(
        _sc_cols,
        out_type=jax.ShapeDtypeStruct((_B, _P), f32),
        mesh=plsc.VectorSubcoreMesh(core_axis_name="c", subcore_axis_name="s",
                                    num_cores=2, num_subcores=16),
        scratch_types=[
            pltpu.VMEM((2 * _BPW, 16), jnp.int32),
            pltpu.VMEM((2 * _BPW, _PPAD), jnp.int32),
            pltpu.VMEM((2 * _BPW, _PPAD), f32),
            pltpu.VMEM((_BPW, _P), f32),
            pltpu.VMEM((_BPW, _P), f32),
            pltpu.SemaphoreType.DMA,
        ],
    )(pred_w.reshape(-1), cols_pre, pred_dense)

    return (new_u, prev_u, new_i, predicted, prev_i, new_umem, new_imem)


# speedup vs baseline: 1.5641x; 1.0447x over previous
import functools

import jax
import jax.numpy as jnp
from jax import lax
from jax.experimental import pallas as pl
from jax.experimental.pallas import tpu as pltpu
from jax.experimental.pallas import tpu_sc as plsc

_B = 128
_NU = 10000
_NI = 2000
_D = 64
_P = _NI + _D
_W = _D + _NU + _D + _NI
_ITEM_BLK0 = 9984
_PPAD = 2176
_NW = 32
_BPW = _B // _NW


def _compute(uid_ref, iid_ref, uf_ref, if_ref, umem_ref, imem_ref,
             uw_ref, uwl_ref, ub_ref, iw_ref, iwl_ref, ib_ref,
             twt_ref, tb_ref, pw_ref, pb_ref,
             new_u_ref, prev_u_ref, new_i_ref, prev_i_ref, pd_ref,
             pu_s, pi_s, wu_s, wi_s, sem_gu, sem_gi, sem_w):
    gu = []
    gi = []
    for b in range(_B):
        u = uid_ref[b]
        i = iid_ref[b]
        cu = pltpu.make_async_copy(
            umem_ref.at[b].at[pl.ds(u, 1), :], pu_s.at[pl.ds(b, 1), :], sem_gu)
        ci = pltpu.make_async_copy(
            imem_ref.at[b].at[pl.ds(i, 1), :], pi_s.at[pl.ds(b, 1), :], sem_gi)
        cu.start()
        ci.start()
        gu.append(cu)
        gi.append(ci)

    wcu = pltpu.make_async_copy(pw_ref.at[:, pl.ds(0, 128)], wu_s, sem_w)
    wci = pltpu.make_async_copy(pw_ref.at[:, pl.ds(_ITEM_BLK0, 256)], wi_s,
                                sem_w)
    wcu.start()
    wci.start()

    for c in gu:
        c.wait()
    for c in gi:
        c.wait()

    prev_u = pu_s[...]
    prev_i = pi_s[...]
    prev_u_ref[...] = prev_u
    prev_i_ref[...] = prev_i

    uf = uf_ref[...]
    itf = if_ref[...]
    time_context = uf * twt_ref[...] + tb_ref[...]
    user_proj = (1.0 + time_context) * prev_u

    f32 = jnp.float32
    dn = (((1,), (1,)), ((), ()))
    uw = uw_ref[...]
    iw = iw_ref[...]
    u_pre = (lax.dot_general(prev_u, uw[:, 0:_D], dn, preferred_element_type=f32)
             + lax.dot_general(prev_i, uw[:, _D:2 * _D], dn,
                               preferred_element_type=f32)
             + uf * uwl_ref[...] + ub_ref[...])
    i_pre = (lax.dot_general(prev_i, iw[:, 0:_D], dn, preferred_element_type=f32)
             + lax.dot_general(prev_u, iw[:, _D:2 * _D], dn,
                               preferred_element_type=f32)
             + itf * iwl_ref[...] + ib_ref[...])
    new_u_ref[...] = jax.nn.sigmoid(u_pre)
    new_i_ref[...] = jax.nn.sigmoid(i_pre)

    wcu.wait()
    wci.wait()
    pd_ref[...] = (
        lax.dot_general(user_proj, wu_s[...][:, 0:_D], dn,
                        preferred_element_type=f32)
        + lax.dot_general(prev_i, wi_s[...][:, 80:144], dn,
                          preferred_element_type=f32)
        + pb_ref[...])


def _copy_scatter(ids_ref, mem_ref, new_ref, out_ref):
    b = pl.program_id(0)
    out_ref[...] = mem_ref[...]
    rid = ids_ref[b]
    r0 = rid // 2
    lane_s = (rid % 2) * _D
    a8 = pl.multiple_of(r0 - r0 % 8, 8)
    sub = r0 % 8
    row = new_ref[b, :]
    doubled = jnp.concatenate([row, row])
    i32 = jnp.int32
    sl = lax.broadcasted_iota(i32, (8, 128), 0)
    ln = lax.broadcasted_iota(i32, (8, 128), 1)
    mask = (sl == sub) & (ln >= lane_s) & (ln < lane_s + _D)
    seg = out_ref[0, pl.ds(a8, 8), :]
    val = jnp.broadcast_to(doubled[None, :], (8, 128))
    out_ref[0, pl.ds(a8, 8), :] = jnp.where(mask, val, seg)


def _scatter_copy_call(mem, ids, new_rows, n):
    f32 = jnp.float32
    nr = n * _D // 128
    flat = pl.pallas_call(
        _copy_scatter,
        grid=(_B,),
        in_specs=[
            pl.BlockSpec(memory_space=pltpu.MemorySpace.SMEM),
            pl.BlockSpec((1, nr, 128), lambda b: (b, 0, 0)),
            pl.BlockSpec((_B, _D), lambda b: (0, 0)),
        ],
        out_specs=pl.BlockSpec((1, nr, 128), lambda b: (b, 0, 0)),
        out_shape=jax.ShapeDtypeStruct((_B, nr, 128), f32),
    )(ids, mem.reshape(_B, nr, 128), new_rows)
    return flat.reshape(_B, n, _D)


def _sc_cols(pwflat_ref, cols_ref, pd_ref, out_ref,
             cols_v, pd_v, out_v, sem, *kv):
    i32 = jnp.int32
    idx_vs = kv[:2 * _BPW]
    vals_vs = kv[2 * _BPW:]
    wid = lax.axis_index("c") * 16 + lax.axis_index("s")
    b0 = wid * _BPW
    pltpu.sync_copy(cols_ref.at[pl.ds(2 * b0, 2 * _BPW)], cols_v)
    pltpu.sync_copy(pd_ref.at[pl.ds(b0, _BPW)], pd_v)

    for k in range(2 * _BPW):
        col = cols_v[k]

        def fill(t, _, k=k, col=col):
            jv = lax.iota(i32, 16) + 16 * t
            idx_vs[k][pl.ds(16 * t, 16)] = jv * _W + col
            return 0

        lax.fori_loop(0, _P // 16, fill, 0)

    copies = []
    for k in range(2 * _BPW):
        c = pltpu.make_async_copy(
            pwflat_ref.at[idx_vs[k]], vals_vs[k], sem)
        c.start()
        copies.append(c)
    for c in copies:
        c.wait()

    for bl in range(_BPW):
        def acc(t, _, bl=bl):
            o = pl.ds(16 * t, 16)
            out_v[bl, o] = (pd_v[bl, o] + vals_vs[2 * bl][o]
                            + vals_vs[2 * bl + 1][o])
            return 0

        lax.fori_loop(0, _P // 16, acc, 0)
    pltpu.sync_copy(out_v, out_ref.at[pl.ds(b0, _BPW)])


def kernel(user_ids, item_ids, user_features, item_features, user_memory,
           item_memory, user_rnn_w, user_rnn_b, item_rnn_w, item_rnn_b,
           time_w, time_b, pred_w, pred_b):
    f32 = jnp.float32
    smem = pl.BlockSpec(memory_space=pltpu.MemorySpace.SMEM)
    vmem = pl.BlockSpec(memory_space=pltpu.MemorySpace.VMEM)
    hbm = pl.BlockSpec(memory_space=pltpu.MemorySpace.HBM)

    uwl = user_rnn_w[:, 2 * _D].reshape(1, _D)
    iwl = item_rnn_w[:, 2 * _D].reshape(1, _D)
    twt = time_w.reshape(1, _D)
    tb2 = time_b.reshape(1, _D)
    ub2 = user_rnn_b.reshape(1, _D)
    ib2 = item_rnn_b.reshape(1, _D)
    pb2 = pred_b.reshape(1, _P)

    new_u, prev_u, new_i, prev_i, pred_dense = pl.pallas_call(
        _compute,
        grid_spec=pltpu.PrefetchScalarGridSpec(
            num_scalar_prefetch=0,
            in_specs=[smem, smem, vmem, vmem, hbm, hbm,
                      vmem, vmem, vmem, vmem, vmem, vmem, vmem, vmem,
                      hbm, vmem],
            out_specs=[vmem, vmem, vmem, vmem, vmem],
            scratch_shapes=[
                pltpu.VMEM((_B, _D), f32),
                pltpu.VMEM((_B, _D), f32),
                pltpu.VMEM((_P, 128), f32),
                pltpu.VMEM((_P, 256), f32),
                pltpu.SemaphoreType.DMA,
                pltpu.SemaphoreType.DMA,
                pltpu.SemaphoreType.DMA,
            ],
        ),
        out_shape=(
            jax.ShapeDtypeStruct((_B, _D), f32),
            jax.ShapeDtypeStruct((_B, _D), f32),
            jax.ShapeDtypeStruct((_B, _D), f32),
            jax.ShapeDtypeStruct((_B, _D), f32),
            jax.ShapeDtypeStruct((_B, _P), f32),
        ),
    )(user_ids, item_ids, user_features, item_features, user_memory,
      item_memory, user_rnn_w, uwl, ub2, item_rnn_w, iwl, ib2, twt, tb2,
      pred_w, pb2)

    new_umem = _scatter_copy_call(user_memory, user_ids, new_u, _NU)
    new_imem = _scatter_copy_call(item_memory, item_ids, new_i, _NI)

    colvals = jnp.stack([user_ids + _D, item_ids + (2 * _D + _NU)],
                        axis=1).reshape(2 * _B)
    cols_pre = jnp.broadcast_to(colvals[:, None], (2 * _B, 16))

    predicted = pl.kernel(
        _sc_cols,
        out_type=jax.ShapeDtypeStruct((_B, _P), f32),
        mesh=plsc.VectorSubcoreMesh(core_axis_name="c", subcore_axis_name="s",
                                    num_cores=2, num_subcores=16),
        scratch_types=[
            pltpu.VMEM((2 * _BPW, 16), jnp.int32),
            pltpu.VMEM((_BPW, _P), f32),
            pltpu.VMEM((_BPW, _P), f32),
            pltpu.SemaphoreType.DMA,
        ] + [pltpu.VMEM((_P,), jnp.int32) for _ in range(2 * _BPW)]
          + [pltpu.VMEM((_P,), f32) for _ in range(2 * _BPW)],
    )(pred_w.reshape(-1), cols_pre, pred_dense)

    return (new_u, prev_u, new_i, predicted, prev_i, new_umem, new_imem)
